# Initial kernel scaffold; baseline (speedup 1.0000x reference)
import functools

import jax
import jax.numpy as jnp
from jax import lax
from jax.experimental import pallas as pl
from jax.experimental.pallas import tpu as pltpu
from jax.experimental.pallas import tpu_sc as plsc

N = 50000
E = 800000
IN_DIM = 64
FD = 128
FC = 32
NP = 50176
EP = 802816
ROWS_E = EP // 128
RPW_H = ROWS_E // 32
RPW_S = ROWS_E // 16
RPT = NP // 16
BR = 1568

_mesh = plsc.VectorSubcoreMesh(core_axis_name="c", subcore_axis_name="s")


@functools.partial(
    pl.kernel,
    out_type=jax.ShapeDtypeStruct((2, NP), jnp.float32),
    mesh=_mesh,
    scratch_types=[
        pltpu.VMEM((RPW_H, 128), jnp.int32),
        pltpu.VMEM((NP,), jnp.float32),
        pltpu.VMEM_SHARED((16, NP), jnp.float32),
        pltpu.VMEM((RPT,), jnp.float32),
        pltpu.VMEM((RPT,), jnp.float32),
    ],
)
def _deg_kernel(dst_hbm, out_hbm, dstv, hist, shared, racc, rtmp):
    c = lax.axis_index("c")
    s = lax.axis_index("s")
    w = c * 16 + s

    zero16 = jnp.zeros((16,), jnp.float32)

    def zbody(i, _):
        hist[pl.ds(i * 16, 16)] = zero16
        return 0

    lax.fori_loop(0, NP // 16, zbody, 0)

    pltpu.sync_copy(dst_hbm.at[pl.ds(w * RPW_H, RPW_H)], dstv)

    ones16 = jnp.ones((16,), jnp.float32)

    def hbody(r, _):
        for j in range(8):
            idx = dstv[r, pl.ds(j * 16, 16)]
            plsc.addupdate_scatter(hist, [idx], ones16)
        return 0

    lax.fori_loop(0, RPW_H, hbody, 0)

    pltpu.sync_copy(hist, shared.at[s])
    plsc.subcore_barrier()

    base = s * RPT
    pltpu.sync_copy(shared.at[0, pl.ds(base, RPT)], racc)
    for t in range(1, 16):
        pltpu.sync_copy(shared.at[t, pl.ds(base, RPT)], rtmp)

        def abody(i, _):
            sl = pl.ds(i * 16, 16)
            racc[sl] = racc[sl] + rtmp[sl]
            return 0

        lax.fori_loop(0, RPT // 16, abody, 0)

    pltpu.sync_copy(racc, out_hbm.at[c, pl.ds(base, RPT)])


@functools.partial(
    pl.kernel,
    out_type=[jax.ShapeDtypeStruct((NP, FC), jnp.float32)] * 4,
    mesh=_mesh,
    scratch_types=[
        pltpu.VMEM((RPW_S, 128), jnp.int32),
        pltpu.VMEM((RPW_S, 128), jnp.int32),
        pltpu.VMEM((128, FC), jnp.float32),
        pltpu.VMEM_SHARED((NP, FC), jnp.float32),
        pltpu.SemaphoreType.DMA,
    ],
)
def _spmm_kernel(y0, y1, y2, y3, src_hbm, dst_hbm, o0, o1, o2, o3,
                 srcv, dstv, rows, acc, gsem):
    c = lax.axis_index("c")
    s = lax.axis_index("s")

    pltpu.sync_copy(src_hbm.at[pl.ds(s * RPW_S, RPW_S)], srcv)
    pltpu.sync_copy(dst_hbm.at[pl.ds(s * RPW_S, RPW_S)], dstv)

    ys = (y0, y1, y2, y3)
    os_ = (o0, o1, o2, o3)
    base = s * RPT

    def process(y_ref, out_ref):
        pltpu.sync_copy(y_ref.at[pl.ds(base, RPT)], acc.at[pl.ds(base, RPT)])
        plsc.subcore_barrier()

        def body(b, _):
            pltpu.async_copy(y_ref.at[srcv.at[b]], rows, gsem).wait()
            pltpu.sync_copy(rows, acc.at[dstv.at[b]], add=True)
            return 0

        lax.fori_loop(0, RPW_S, body, 0)
        plsc.subcore_barrier()
        pltpu.sync_copy(acc.at[pl.ds(base, RPT)], out_ref.at[pl.ds(base, RPT)])
        plsc.subcore_barrier()

    for cid in range(2):
        @pl.when(c == cid)
        def _():
            process(ys[2 * cid], os_[2 * cid])
            process(ys[2 * cid + 1], os_[2 * cid + 1])


def _dvec(degt_ref):
    deg = degt_ref[:, 0:1] + degt_ref[:, 1:2] + 1.0
    return lax.rsqrt(deg)


def _tc1_body(degt_ref, z_ref, w1_ref, y0_ref, y1_ref, y2_ref, y3_ref):
    d = _dvec(degt_ref)
    xw = jnp.dot(z_ref[...], w1_ref[...], preferred_element_type=jnp.float32)
    y = xw * d
    y0_ref[...] = y[:, 0:32]
    y1_ref[...] = y[:, 32:64]
    y2_ref[...] = y[:, 64:96]
    y3_ref[...] = y[:, 96:128]


def _tc2_body(degt_ref, s0_ref, s1_ref, s2_ref, s3_ref, b1_ref, w2_ref,
              y0_ref, y1_ref, y2_ref, y3_ref):
    d = _dvec(degt_ref)
    S = jnp.concatenate(
        [s0_ref[...], s1_ref[...], s2_ref[...], s3_ref[...]], axis=1)
    h = jnp.maximum(S * d + b1_ref[...], 0.0)
    xw = jnp.dot(h, w2_ref[...], preferred_element_type=jnp.float32)
    y = xw * d
    y0_ref[...] = y[:, 0:32]
    y1_ref[...] = y[:, 32:64]
    y2_ref[...] = y[:, 64:96]
    y3_ref[...] = y[:, 96:128]


def _tc3_body(degt_ref, s0_ref, s1_ref, s2_ref, s3_ref, b2_ref, out_ref):
    d = _dvec(degt_ref)
    S = jnp.concatenate(
        [s0_ref[...], s1_ref[...], s2_ref[...], s3_ref[...]], axis=1)
    out_ref[...] = jnp.maximum(S * d + b2_ref[...], 0.0)


_GRID = (NP // BR,)
_bs_degt = pl.BlockSpec((BR, 2), lambda i: (i, 0))
_bs_z = pl.BlockSpec((BR, IN_DIM), lambda i: (i, 0))
_bs_w1 = pl.BlockSpec((IN_DIM, FD), lambda i: (0, 0))
_bs_w2 = pl.BlockSpec((FD, FD), lambda i: (0, 0))
_bs_b = pl.BlockSpec((1, FD), lambda i: (0, 0))
_bs_c = pl.BlockSpec((BR, FC), lambda i: (i, 0))
_bs_f = pl.BlockSpec((BR, FD), lambda i: (i, 0))

_chunk4 = [jax.ShapeDtypeStruct((NP, FC), jnp.float32)] * 4

_tc1 = pl.pallas_call(
    _tc1_body, grid=_GRID,
    in_specs=[_bs_degt, _bs_z, _bs_w1],
    out_specs=[_bs_c] * 4,
    out_shape=_chunk4,
)

_tc2 = pl.pallas_call(
    _tc2_body, grid=_GRID,
    in_specs=[_bs_degt, _bs_c, _bs_c, _bs_c, _bs_c, _bs_b, _bs_w2],
    out_specs=[_bs_c] * 4,
    out_shape=_chunk4,
)

_tc3 = pl.pallas_call(
    _tc3_body, grid=_GRID,
    in_specs=[_bs_degt, _bs_c, _bs_c, _bs_c, _bs_c, _bs_b],
    out_specs=_bs_f,
    out_shape=jax.ShapeDtypeStruct((NP, FD), jnp.float32),
)


def kernel(z, edge_index, W1, b1, W2, b2):
    src = jnp.concatenate(
        [edge_index[0], jnp.zeros((EP - E,), jnp.int32)]).reshape(ROWS_E, 128)
    dst = jnp.concatenate(
        [edge_index[1], jnp.full((EP - E,), N, jnp.int32)]).reshape(ROWS_E, 128)
    z_pad = jnp.pad(z, ((0, NP - N), (0, 0)))

    degp = _deg_kernel(dst)
    degt = degp.T

    y1 = _tc1(degt, z_pad, W1)
    s1 = _spmm_kernel(*y1, src, dst)
    y2 = _tc2(degt, *s1, b1.reshape(1, FD), W2)
    s2 = _spmm_kernel(*y2, src, dst)
    xp = _tc3(degt, *s2, b2.reshape(1, FD))
    return xp[:N]


# speedup vs baseline: 8.0121x; 8.0121x over previous
import functools

import jax
import jax.numpy as jnp
from jax import lax
from jax.experimental import pallas as pl
from jax.experimental.pallas import tpu as pltpu
from jax.experimental.pallas import tpu_sc as plsc

N = 50000
E = 800000
IN_DIM = 64
FD = 128
FC = 32
NP = 50176
EP = 819200
ROWS_E = EP // 128
RPW_H = ROWS_E // 32
RPW_S = ROWS_E // 16
RPT = NP // 16
SB = 16
BR = 1568

_mesh = plsc.VectorSubcoreMesh(core_axis_name="c", subcore_axis_name="s")


@functools.partial(
    pl.kernel,
    out_type=jax.ShapeDtypeStruct((2 * NP,), jnp.float32),
    mesh=_mesh,
    compiler_params=pltpu.CompilerParams(needs_layout_passes=False),
    scratch_types=[
        pltpu.VMEM((40, 128), jnp.int32),
        pltpu.VMEM((NP,), jnp.float32),
        pltpu.VMEM_SHARED((16 * NP,), jnp.float32),
        pltpu.VMEM((RPT,), jnp.float32),
        pltpu.VMEM((RPT,), jnp.float32),
    ],
)
def _deg_kernel(dst_hbm, out_hbm, dstv, hist, shared, racc, rtmp):
    c = lax.axis_index("c")
    s = lax.axis_index("s")
    w = c * 16 + s

    zero16 = jnp.zeros((16,), jnp.float32)

    def zbody(i, _):
        hist[pl.ds(i * 16, 16)] = zero16
        return 0

    lax.fori_loop(0, NP // 16, zbody, 0)

    ones16 = jnp.ones((16,), jnp.float32)

    def kbody(k, _):
        pltpu.sync_copy(dst_hbm.at[pl.ds(w * RPW_H + k * 40, 40)], dstv)

        def hbody(r, _):
            for j in range(8):
                idx = dstv[r, pl.ds(j * 16, 16)]
                plsc.addupdate_scatter(hist, [idx], ones16)
            return 0

        lax.fori_loop(0, 40, hbody, 0)
        return 0

    lax.fori_loop(0, RPW_H // 40, kbody, 0)

    pltpu.sync_copy(hist, shared.at[pl.ds(s * NP, NP)])
    plsc.subcore_barrier()

    base = s * RPT
    pltpu.sync_copy(shared.at[pl.ds(base, RPT)], racc)
    for t in range(1, 16):
        pltpu.sync_copy(shared.at[pl.ds(t * NP + base, RPT)], rtmp)

        def abody(i, _):
            sl = pl.ds(i * 16, 16)
            racc[sl] = racc[sl] + rtmp[sl]
            return 0

        lax.fori_loop(0, RPT // 16, abody, 0)

    pltpu.sync_copy(racc, out_hbm.at[pl.ds(c * NP + base, RPT)])


@functools.partial(
    pl.kernel,
    out_type=[jax.ShapeDtypeStruct((NP, FC), jnp.float32)] * 4,
    mesh=_mesh,
    compiler_params=pltpu.CompilerParams(
        needs_layout_passes=False, use_tc_tiling_on_sc=False),
    scratch_types=[
        pltpu.VMEM((SB, 128), jnp.int32),
        pltpu.VMEM((SB, 128), jnp.int32),
        pltpu.VMEM((128, FC), jnp.float32),
        pltpu.VMEM_SHARED((NP, FC), jnp.float32),
        pltpu.SemaphoreType.DMA,
    ],
)
def _spmm_kernel(y0, y1, y2, y3, src_hbm, dst_hbm, o0, o1, o2, o3,
                 srcv, dstv, rows, acc, gsem):
    c = lax.axis_index("c")
    s = lax.axis_index("s")

    ys = (y0, y1, y2, y3)
    os_ = (o0, o1, o2, o3)
    base = s * RPT

    def process(y_ref, out_ref):
        pltpu.sync_copy(y_ref.at[pl.ds(base, RPT)], acc.at[pl.ds(base, RPT)])
        plsc.subcore_barrier()

        def sbody(sb, _):
            r0 = s * RPW_S + sb * SB
            pltpu.sync_copy(src_hbm.at[pl.ds(r0, SB)], srcv)
            pltpu.sync_copy(dst_hbm.at[pl.ds(r0, SB)], dstv)

            def body(b, _):
                pltpu.async_copy(y_ref.at[srcv.at[b]], rows, gsem).wait()
                pltpu.sync_copy(rows, acc.at[dstv.at[b]], add=True)
                return 0

            lax.fori_loop(0, SB, body, 0)
            return 0

        lax.fori_loop(0, RPW_S // SB, sbody, 0)
        plsc.subcore_barrier()
        pltpu.sync_copy(acc.at[pl.ds(base, RPT)], out_ref.at[pl.ds(base, RPT)])
        plsc.subcore_barrier()

    for cid in range(2):
        @pl.when(c == cid)
        def _():
            process(ys[2 * cid], os_[2 * cid])
            process(ys[2 * cid + 1], os_[2 * cid + 1])


def _dvec(degt_ref):
    deg = degt_ref[:, 0:1] + degt_ref[:, 1:2] + 1.0
    return lax.rsqrt(deg)


def _tc1_body(degt_ref, z_ref, w1_ref, y0_ref, y1_ref, y2_ref, y3_ref):
    d = _dvec(degt_ref)
    xw = jnp.dot(z_ref[...], w1_ref[...], preferred_element_type=jnp.float32)
    y = xw * d
    y0_ref[...] = y[:, 0:32]
    y1_ref[...] = y[:, 32:64]
    y2_ref[...] = y[:, 64:96]
    y3_ref[...] = y[:, 96:128]


def _tc2_body(degt_ref, s0_ref, s1_ref, s2_ref, s3_ref, b1_ref, w2_ref,
              y0_ref, y1_ref, y2_ref, y3_ref):
    d = _dvec(degt_ref)
    S = jnp.concatenate(
        [s0_ref[...], s1_ref[...], s2_ref[...], s3_ref[...]], axis=1)
    h = jnp.maximum(S * d + b1_ref[...], 0.0)
    xw = jnp.dot(h, w2_ref[...], preferred_element_type=jnp.float32)
    y = xw * d
    y0_ref[...] = y[:, 0:32]
    y1_ref[...] = y[:, 32:64]
    y2_ref[...] = y[:, 64:96]
    y3_ref[...] = y[:, 96:128]


def _tc3_body(degt_ref, s0_ref, s1_ref, s2_ref, s3_ref, b2_ref, out_ref):
    d = _dvec(degt_ref)
    S = jnp.concatenate(
        [s0_ref[...], s1_ref[...], s2_ref[...], s3_ref[...]], axis=1)
    out_ref[...] = jnp.maximum(S * d + b2_ref[...], 0.0)


_GRID = (NP // BR,)
_bs_degt = pl.BlockSpec((BR, 2), lambda i: (i, 0))
_bs_z = pl.BlockSpec((BR, IN_DIM), lambda i: (i, 0))
_bs_w1 = pl.BlockSpec((IN_DIM, FD), lambda i: (0, 0))
_bs_w2 = pl.BlockSpec((FD, FD), lambda i: (0, 0))
_bs_b = pl.BlockSpec((1, FD), lambda i: (0, 0))
_bs_c = pl.BlockSpec((BR, FC), lambda i: (i, 0))
_bs_f = pl.BlockSpec((BR, FD), lambda i: (i, 0))

_chunk4 = [jax.ShapeDtypeStruct((NP, FC), jnp.float32)] * 4

_tc1 = pl.pallas_call(
    _tc1_body, grid=_GRID,
    in_specs=[_bs_degt, _bs_z, _bs_w1],
    out_specs=[_bs_c] * 4,
    out_shape=_chunk4,
)

_tc2 = pl.pallas_call(
    _tc2_body, grid=_GRID,
    in_specs=[_bs_degt, _bs_c, _bs_c, _bs_c, _bs_c, _bs_b, _bs_w2],
    out_specs=[_bs_c] * 4,
    out_shape=_chunk4,
)

_tc3 = pl.pallas_call(
    _tc3_body, grid=_GRID,
    in_specs=[_bs_degt, _bs_c, _bs_c, _bs_c, _bs_c, _bs_b],
    out_specs=_bs_f,
    out_shape=jax.ShapeDtypeStruct((NP, FD), jnp.float32),
)


def kernel(z, edge_index, W1, b1, W2, b2):
    src = jnp.concatenate(
        [edge_index[0], jnp.zeros((EP - E,), jnp.int32)]).reshape(ROWS_E, 128)
    dst = jnp.concatenate(
        [edge_index[1], jnp.full((EP - E,), N, jnp.int32)]).reshape(ROWS_E, 128)
    z_pad = jnp.pad(z, ((0, NP - N), (0, 0)))

    degt = _deg_kernel(dst).reshape(2, NP).T

    y1 = _tc1(degt, z_pad, W1)
    s1 = _spmm_kernel(*y1, src, dst)
    y2 = _tc2(degt, *s1, b1.reshape(1, FD), W2)
    s2 = _spmm_kernel(*y2, src, dst)
    xp = _tc3(degt, *s2, b2.reshape(1, FD))
    return xp[:N]


# speedup vs baseline: 10.5497x; 1.3167x over previous
import functools

import jax
import jax.numpy as jnp
from jax import lax
from jax.experimental import pallas as pl
from jax.experimental.pallas import tpu as pltpu
from jax.experimental.pallas import tpu_sc as plsc

N = 50000
E = 800000
IN_DIM = 64
FD = 128
FC = 32
NP = 50176
EP = 819200
ROWS_E = EP // 128
RPW_H = ROWS_E // 32
RPW_S = ROWS_E // 16
RPT = NP // 16
SB = 8
KSLOT = 4
BR = 1568

_mesh = plsc.VectorSubcoreMesh(core_axis_name="c", subcore_axis_name="s")


@functools.partial(
    pl.kernel,
    out_type=jax.ShapeDtypeStruct((2 * NP,), jnp.float32),
    mesh=_mesh,
    compiler_params=pltpu.CompilerParams(needs_layout_passes=False),
    scratch_types=[
        pltpu.VMEM((40, 128), jnp.int32),
        pltpu.VMEM((NP,), jnp.float32),
        pltpu.VMEM_SHARED((16 * NP,), jnp.float32),
        pltpu.VMEM((RPT,), jnp.float32),
        pltpu.VMEM((RPT,), jnp.float32),
    ],
)
def _deg_kernel(dst_hbm, out_hbm, dstv, hist, shared, racc, rtmp):
    c = lax.axis_index("c")
    s = lax.axis_index("s")
    w = c * 16 + s

    zero16 = jnp.zeros((16,), jnp.float32)

    def zbody(i, _):
        hist[pl.ds(i * 16, 16)] = zero16
        return 0

    lax.fori_loop(0, NP // 16, zbody, 0)

    ones16 = jnp.ones((16,), jnp.float32)

    def kbody(k, _):
        pltpu.sync_copy(dst_hbm.at[pl.ds(w * RPW_H + k * 40, 40)], dstv)

        def hbody(r, _):
            for j in range(8):
                idx = dstv[r, pl.ds(j * 16, 16)]
                plsc.addupdate_scatter(hist, [idx], ones16)
            return 0

        lax.fori_loop(0, 40, hbody, 0)
        return 0

    lax.fori_loop(0, RPW_H // 40, kbody, 0)

    pltpu.sync_copy(hist, shared.at[pl.ds(s * NP, NP)])
    plsc.subcore_barrier()

    base = s * RPT
    pltpu.sync_copy(shared.at[pl.ds(base, RPT)], racc)
    for t in range(1, 16):
        pltpu.sync_copy(shared.at[pl.ds(t * NP + base, RPT)], rtmp)

        def abody(i, _):
            sl = pl.ds(i * 16, 16)
            racc[sl] = racc[sl] + rtmp[sl]
            return 0

        lax.fori_loop(0, RPT // 16, abody, 0)

    pltpu.sync_copy(racc, out_hbm.at[pl.ds(c * NP + base, RPT)])


@functools.partial(
    pl.kernel,
    out_type=[jax.ShapeDtypeStruct((NP, FC), jnp.float32)] * 4,
    mesh=_mesh,
    compiler_params=pltpu.CompilerParams(
        needs_layout_passes=False, use_tc_tiling_on_sc=False),
    scratch_types=[
        pltpu.VMEM((SB, 128), jnp.int32),
        pltpu.VMEM((SB, 128), jnp.int32),
        pltpu.VMEM((KSLOT, 128, FC), jnp.float32),
        pltpu.VMEM_SHARED((NP, FC), jnp.float32),
    ] + [pltpu.SemaphoreType.DMA] * KSLOT,
)
def _spmm_kernel(y0, y1, y2, y3, src_hbm, dst_hbm, o0, o1, o2, o3,
                 srcv, dstv, rows, acc, *gsems):
    c = lax.axis_index("c")
    s = lax.axis_index("s")

    ys = (y0, y1, y2, y3)
    os_ = (o0, o1, o2, o3)
    base = s * RPT

    def process(y_ref, out_ref):
        pltpu.sync_copy(y_ref.at[pl.ds(base, RPT)], acc.at[pl.ds(base, RPT)])
        plsc.subcore_barrier()

        def sbody(sb, _):
            r0 = s * RPW_S + sb * SB
            pltpu.sync_copy(src_hbm.at[pl.ds(r0, SB)], srcv)
            pltpu.sync_copy(dst_hbm.at[pl.ds(r0, SB)], dstv)

            descs = [
                pltpu.async_copy(y_ref.at[srcv.at[v]], rows.at[v], gsems[v])
                for v in range(KSLOT)
            ]
            for b in range(SB):
                v = b % KSLOT
                descs[v].wait()
                pltpu.sync_copy(rows.at[v], acc.at[dstv.at[b]], add=True)
                nb = b + KSLOT
                if nb < SB:
                    descs[v] = pltpu.async_copy(
                        y_ref.at[srcv.at[nb]], rows.at[v], gsems[v])
            return 0

        lax.fori_loop(0, RPW_S // SB, sbody, 0)
        plsc.subcore_barrier()
        pltpu.sync_copy(acc.at[pl.ds(base, RPT)], out_ref.at[pl.ds(base, RPT)])
        plsc.subcore_barrier()

    for cid in range(2):
        @pl.when(c == cid)
        def _():
            process(ys[2 * cid], os_[2 * cid])
            process(ys[2 * cid + 1], os_[2 * cid + 1])


def _dvec(degt_ref):
    deg = degt_ref[:, 0:1] + degt_ref[:, 1:2] + 1.0
    return lax.rsqrt(deg)


def _tc1_body(degt_ref, z_ref, w1_ref, y0_ref, y1_ref, y2_ref, y3_ref):
    d = _dvec(degt_ref)
    xw = jnp.dot(z_ref[...], w1_ref[...], preferred_element_type=jnp.float32)
    y = xw * d
    y0_ref[...] = y[:, 0:32]
    y1_ref[...] = y[:, 32:64]
    y2_ref[...] = y[:, 64:96]
    y3_ref[...] = y[:, 96:128]


def _tc2_body(degt_ref, s0_ref, s1_ref, s2_ref, s3_ref, b1_ref, w2_ref,
              y0_ref, y1_ref, y2_ref, y3_ref):
    d = _dvec(degt_ref)
    S = jnp.concatenate(
        [s0_ref[...], s1_ref[...], s2_ref[...], s3_ref[...]], axis=1)
    h = jnp.maximum(S * d + b1_ref[...], 0.0)
    xw = jnp.dot(h, w2_ref[...], preferred_element_type=jnp.float32)
    y = xw * d
    y0_ref[...] = y[:, 0:32]
    y1_ref[...] = y[:, 32:64]
    y2_ref[...] = y[:, 64:96]
    y3_ref[...] = y[:, 96:128]


def _tc3_body(degt_ref, s0_ref, s1_ref, s2_ref, s3_ref, b2_ref, out_ref):
    d = _dvec(degt_ref)
    S = jnp.concatenate(
        [s0_ref[...], s1_ref[...], s2_ref[...], s3_ref[...]], axis=1)
    out_ref[...] = jnp.maximum(S * d + b2_ref[...], 0.0)


_GRID = (NP // BR,)
_bs_degt = pl.BlockSpec((BR, 2), lambda i: (i, 0))
_bs_z = pl.BlockSpec((BR, IN_DIM), lambda i: (i, 0))
_bs_w1 = pl.BlockSpec((IN_DIM, FD), lambda i: (0, 0))
_bs_w2 = pl.BlockSpec((FD, FD), lambda i: (0, 0))
_bs_b = pl.BlockSpec((1, FD), lambda i: (0, 0))
_bs_c = pl.BlockSpec((BR, FC), lambda i: (i, 0))
_bs_f = pl.BlockSpec((BR, FD), lambda i: (i, 0))

_chunk4 = [jax.ShapeDtypeStruct((NP, FC), jnp.float32)] * 4

_tc1 = pl.pallas_call(
    _tc1_body, grid=_GRID,
    in_specs=[_bs_degt, _bs_z, _bs_w1],
    out_specs=[_bs_c] * 4,
    out_shape=_chunk4,
)

_tc2 = pl.pallas_call(
    _tc2_body, grid=_GRID,
    in_specs=[_bs_degt, _bs_c, _bs_c, _bs_c, _bs_c, _bs_b, _bs_w2],
    out_specs=[_bs_c] * 4,
    out_shape=_chunk4,
)

_tc3 = pl.pallas_call(
    _tc3_body, grid=_GRID,
    in_specs=[_bs_degt, _bs_c, _bs_c, _bs_c, _bs_c, _bs_b],
    out_specs=_bs_f,
    out_shape=jax.ShapeDtypeStruct((NP, FD), jnp.float32),
)


def kernel(z, edge_index, W1, b1, W2, b2):
    src = jnp.concatenate(
        [edge_index[0], jnp.zeros((EP - E,), jnp.int32)]).reshape(ROWS_E, 128)
    dst = jnp.concatenate(
        [edge_index[1], jnp.full((EP - E,), N, jnp.int32)]).reshape(ROWS_E, 128)
    z_pad = jnp.pad(z, ((0, NP - N), (0, 0)))

    degt = _deg_kernel(dst).reshape(2, NP).T

    y1 = _tc1(degt, z_pad, W1)
    s1 = _spmm_kernel(*y1, src, dst)
    y2 = _tc2(degt, *s1, b1.reshape(1, FD), W2)
    s2 = _spmm_kernel(*y2, src, dst)
    xp = _tc3(degt, *s2, b2.reshape(1, FD))
    return xp[:N]


# speedup vs baseline: 11.5539x; 1.0952x over previous
import functools

import jax
import jax.numpy as jnp
from jax import lax
from jax.experimental import pallas as pl
from jax.experimental.pallas import tpu as pltpu
from jax.experimental.pallas import tpu_sc as plsc

N = 50000
E = 800000
IN_DIM = 64
FD = 128
FC = 32
NP = 50176
EP = 819200
ROWS_E = EP // 128
RPW_H = ROWS_E // 32
RPW_S = ROWS_E // 16
RPT = NP // 16
SB = 8
KSLOT = 4
BR = 1568

_mesh = plsc.VectorSubcoreMesh(core_axis_name="c", subcore_axis_name="s")


@functools.partial(
    pl.kernel,
    out_type=jax.ShapeDtypeStruct((2 * NP,), jnp.float32),
    mesh=_mesh,
    compiler_params=pltpu.CompilerParams(needs_layout_passes=False),
    scratch_types=[
        pltpu.VMEM((40, 128), jnp.int32),
        pltpu.VMEM((NP,), jnp.float32),
        pltpu.VMEM_SHARED((16 * NP,), jnp.float32),
        pltpu.VMEM((RPT,), jnp.float32),
        pltpu.VMEM((RPT,), jnp.float32),
    ],
)
def _deg_kernel(dst_hbm, out_hbm, dstv, hist, shared, racc, rtmp):
    c = lax.axis_index("c")
    s = lax.axis_index("s")
    w = c * 16 + s

    zero16 = jnp.zeros((16,), jnp.float32)

    def zbody(i, _):
        hist[pl.ds(i * 16, 16)] = zero16
        return 0

    lax.fori_loop(0, NP // 16, zbody, 0)

    ones16 = jnp.ones((16,), jnp.float32)

    def kbody(k, _):
        pltpu.sync_copy(dst_hbm.at[pl.ds(w * RPW_H + k * 40, 40)], dstv)

        def hbody(r, _):
            for j in range(8):
                idx = dstv[r, pl.ds(j * 16, 16)]
                plsc.addupdate_scatter(hist, [idx], ones16)
            return 0

        lax.fori_loop(0, 40, hbody, 0)
        return 0

    lax.fori_loop(0, RPW_H // 40, kbody, 0)

    pltpu.sync_copy(hist, shared.at[pl.ds(s * NP, NP)])
    plsc.subcore_barrier()

    base = s * RPT
    pltpu.sync_copy(shared.at[pl.ds(base, RPT)], racc)
    for t in range(1, 16):
        pltpu.sync_copy(shared.at[pl.ds(t * NP + base, RPT)], rtmp)

        def abody(i, _):
            sl = pl.ds(i * 16, 16)
            racc[sl] = racc[sl] + rtmp[sl]
            return 0

        lax.fori_loop(0, RPT // 16, abody, 0)

    pltpu.sync_copy(racc, out_hbm.at[pl.ds(c * NP + base, RPT)])


@functools.partial(
    pl.kernel,
    out_type=[jax.ShapeDtypeStruct((NP, FC), jnp.float32)] * 4,
    mesh=_mesh,
    compiler_params=pltpu.CompilerParams(
        needs_layout_passes=False, use_tc_tiling_on_sc=False),
    scratch_types=[
        pltpu.VMEM((2, SB, 128), jnp.int32),
        pltpu.VMEM((2, SB, 128), jnp.int32),
        pltpu.VMEM((KSLOT, 128, FC), jnp.float32),
        pltpu.VMEM_SHARED((NP, FC), jnp.float32),
    ] + [pltpu.SemaphoreType.DMA] * (2 * KSLOT + 1),
)
def _spmm_kernel(y0, y1, y2, y3, src_hbm, dst_hbm, o0, o1, o2, o3,
                 srcv, dstv, rows, acc, *sems):
    c = lax.axis_index("c")
    s = lax.axis_index("s")
    gsems = sems[:KSLOT]
    ssems = sems[KSLOT:2 * KSLOT]
    isem = sems[2 * KSLOT]

    ys = (y0, y1, y2, y3)
    os_ = (o0, o1, o2, o3)
    base = s * RPT
    NSB = RPW_S // SB

    def idx_copies(sb, buf):
        r0 = s * RPW_S + sb * SB
        return (pltpu.make_async_copy(
                    src_hbm.at[pl.ds(r0, SB)], srcv.at[buf], isem),
                pltpu.make_async_copy(
                    dst_hbm.at[pl.ds(r0, SB)], dstv.at[buf], isem))

    def process(y_ref, out_ref):
        for d in idx_copies(0, 0):
            d.start()
        pltpu.sync_copy(y_ref.at[pl.ds(base, RPT)], acc.at[pl.ds(base, RPT)])
        plsc.subcore_barrier()

        def sbody(sb, _):
            buf = lax.rem(sb, 2)
            for d in idx_copies(sb, buf):
                d.wait()

            @pl.when(sb + 1 < NSB)
            def _():
                for d in idx_copies(sb + 1, 1 - buf):
                    d.start()

            sv = srcv.at[buf]
            dv = dstv.at[buf]
            gdescs = [
                pltpu.async_copy(y_ref.at[sv.at[v]], rows.at[v], gsems[v])
                for v in range(KSLOT)
            ]
            pending = [None] * KSLOT
            for b in range(SB):
                v = b % KSLOT
                if b >= 1:
                    pv = (b - 1) % KSLOT
                    pending[pv].wait()
                    nb = b - 1 + KSLOT
                    if nb < SB:
                        gdescs[pv] = pltpu.async_copy(
                            y_ref.at[sv.at[nb]], rows.at[pv], gsems[pv])
                gdescs[v].wait()
                pending[v] = pltpu.async_copy(
                    rows.at[v], acc.at[dv.at[b]], ssems[v], add=True)
            pending[(SB - 1) % KSLOT].wait()
            return 0

        lax.fori_loop(0, NSB, sbody, 0)
        plsc.subcore_barrier()
        pltpu.sync_copy(acc.at[pl.ds(base, RPT)], out_ref.at[pl.ds(base, RPT)])
        plsc.subcore_barrier()

    for cid in range(2):
        @pl.when(c == cid)
        def _():
            process(ys[2 * cid], os_[2 * cid])
            process(ys[2 * cid + 1], os_[2 * cid + 1])


def _dvec(degt_ref):
    deg = degt_ref[:, 0:1] + degt_ref[:, 1:2] + 1.0
    return lax.rsqrt(deg)


def _tc1_body(degt_ref, z_ref, w1_ref, y0_ref, y1_ref, y2_ref, y3_ref):
    d = _dvec(degt_ref)
    xw = jnp.dot(z_ref[...], w1_ref[...], preferred_element_type=jnp.float32)
    y = xw * d
    y0_ref[...] = y[:, 0:32]
    y1_ref[...] = y[:, 32:64]
    y2_ref[...] = y[:, 64:96]
    y3_ref[...] = y[:, 96:128]


def _tc2_body(degt_ref, s0_ref, s1_ref, s2_ref, s3_ref, b1_ref, w2_ref,
              y0_ref, y1_ref, y2_ref, y3_ref):
    d = _dvec(degt_ref)
    S = jnp.concatenate(
        [s0_ref[...], s1_ref[...], s2_ref[...], s3_ref[...]], axis=1)
    h = jnp.maximum(S * d + b1_ref[...], 0.0)
    xw = jnp.dot(h, w2_ref[...], preferred_element_type=jnp.float32)
    y = xw * d
    y0_ref[...] = y[:, 0:32]
    y1_ref[...] = y[:, 32:64]
    y2_ref[...] = y[:, 64:96]
    y3_ref[...] = y[:, 96:128]


def _tc3_body(degt_ref, s0_ref, s1_ref, s2_ref, s3_ref, b2_ref, out_ref):
    d = _dvec(degt_ref)
    S = jnp.concatenate(
        [s0_ref[...], s1_ref[...], s2_ref[...], s3_ref[...]], axis=1)
    out_ref[...] = jnp.maximum(S * d + b2_ref[...], 0.0)


_GRID = (NP // BR,)
_bs_degt = pl.BlockSpec((BR, 2), lambda i: (i, 0))
_bs_z = pl.BlockSpec((BR, IN_DIM), lambda i: (i, 0))
_bs_w1 = pl.BlockSpec((IN_DIM, FD), lambda i: (0, 0))
_bs_w2 = pl.BlockSpec((FD, FD), lambda i: (0, 0))
_bs_b = pl.BlockSpec((1, FD), lambda i: (0, 0))
_bs_c = pl.BlockSpec((BR, FC), lambda i: (i, 0))
_bs_f = pl.BlockSpec((BR, FD), lambda i: (i, 0))

_chunk4 = [jax.ShapeDtypeStruct((NP, FC), jnp.float32)] * 4

_tc1 = pl.pallas_call(
    _tc1_body, grid=_GRID,
    in_specs=[_bs_degt, _bs_z, _bs_w1],
    out_specs=[_bs_c] * 4,
    out_shape=_chunk4,
)

_tc2 = pl.pallas_call(
    _tc2_body, grid=_GRID,
    in_specs=[_bs_degt, _bs_c, _bs_c, _bs_c, _bs_c, _bs_b, _bs_w2],
    out_specs=[_bs_c] * 4,
    out_shape=_chunk4,
)

_tc3 = pl.pallas_call(
    _tc3_body, grid=_GRID,
    in_specs=[_bs_degt, _bs_c, _bs_c, _bs_c, _bs_c, _bs_b],
    out_specs=_bs_f,
    out_shape=jax.ShapeDtypeStruct((NP, FD), jnp.float32),
)


def kernel(z, edge_index, W1, b1, W2, b2):
    src = jnp.concatenate(
        [edge_index[0], jnp.zeros((EP - E,), jnp.int32)]).reshape(ROWS_E, 128)
    dst = jnp.concatenate(
        [edge_index[1], jnp.full((EP - E,), N, jnp.int32)]).reshape(ROWS_E, 128)
    z_pad = jnp.pad(z, ((0, NP - N), (0, 0)))

    degt = _deg_kernel(dst).reshape(2, NP).T

    y1 = _tc1(degt, z_pad, W1)
    s1 = _spmm_kernel(*y1, src, dst)
    y2 = _tc2(degt, *s1, b1.reshape(1, FD), W2)
    s2 = _spmm_kernel(*y2, src, dst)
    xp = _tc3(degt, *s2, b2.reshape(1, FD))
    return xp[:N]


# speedup vs baseline: 12.0100x; 1.0395x over previous
import functools

import jax
import jax.numpy as jnp
from jax import lax
from jax.experimental import pallas as pl
from jax.experimental.pallas import tpu as pltpu
from jax.experimental.pallas import tpu_sc as plsc

N = 50000
E = 800000
IN_DIM = 64
FD = 128
FC = 32
NP = 50176
EP = 819200
ROWS_E = EP // 128
RPW_H = ROWS_E // 32
RPW_S = ROWS_E // 16
RPT = NP // 16
SB = 16
KSLOT = 5
BR = 1568

_mesh = plsc.VectorSubcoreMesh(core_axis_name="c", subcore_axis_name="s")


@functools.partial(
    pl.kernel,
    out_type=jax.ShapeDtypeStruct((2 * NP,), jnp.float32),
    mesh=_mesh,
    compiler_params=pltpu.CompilerParams(needs_layout_passes=False),
    scratch_types=[
        pltpu.VMEM((40, 128), jnp.int32),
        pltpu.VMEM((NP,), jnp.float32),
        pltpu.VMEM_SHARED((16 * NP,), jnp.float32),
        pltpu.VMEM((RPT,), jnp.float32),
        pltpu.VMEM((RPT,), jnp.float32),
    ],
)
def _deg_kernel(dst_hbm, out_hbm, dstv, hist, shared, racc, rtmp):
    c = lax.axis_index("c")
    s = lax.axis_index("s")
    w = c * 16 + s

    zero16 = jnp.zeros((16,), jnp.float32)

    def zbody(i, _):
        hist[pl.ds(i * 16, 16)] = zero16
        return 0

    lax.fori_loop(0, NP // 16, zbody, 0)

    ones16 = jnp.ones((16,), jnp.float32)

    def kbody(k, _):
        pltpu.sync_copy(dst_hbm.at[pl.ds(w * RPW_H + k * 40, 40)], dstv)

        def hbody(r, _):
            for j in range(8):
                idx = dstv[r, pl.ds(j * 16, 16)]
                plsc.addupdate_scatter(hist, [idx], ones16)
            return 0

        lax.fori_loop(0, 40, hbody, 0)
        return 0

    lax.fori_loop(0, RPW_H // 40, kbody, 0)

    pltpu.sync_copy(hist, shared.at[pl.ds(s * NP, NP)])
    plsc.subcore_barrier()

    base = s * RPT
    pltpu.sync_copy(shared.at[pl.ds(base, RPT)], racc)
    for t in range(1, 16):
        pltpu.sync_copy(shared.at[pl.ds(t * NP + base, RPT)], rtmp)

        def abody(i, _):
            sl = pl.ds(i * 16, 16)
            racc[sl] = racc[sl] + rtmp[sl]
            return 0

        lax.fori_loop(0, RPT // 16, abody, 0)

    pltpu.sync_copy(racc, out_hbm.at[pl.ds(c * NP + base, RPT)])


@functools.partial(
    pl.kernel,
    out_type=[jax.ShapeDtypeStruct((NP, FC), jnp.float32)] * 4,
    mesh=_mesh,
    compiler_params=pltpu.CompilerParams(
        needs_layout_passes=False, use_tc_tiling_on_sc=False),
    scratch_types=[
        pltpu.VMEM((2, SB, 128), jnp.int32),
        pltpu.VMEM((2, SB, 128), jnp.int32),
        pltpu.VMEM((KSLOT, 128, FC), jnp.float32),
        pltpu.VMEM_SHARED((NP, FC), jnp.float32),
    ] + [pltpu.SemaphoreType.DMA] * (2 * KSLOT + 1),
)
def _spmm_kernel(y0, y1, y2, y3, src_hbm, dst_hbm, o0, o1, o2, o3,
                 srcv, dstv, rows, acc, *sems):
    c = lax.axis_index("c")
    s = lax.axis_index("s")
    gsems = sems[:KSLOT]
    ssems = sems[KSLOT:2 * KSLOT]
    isem = sems[2 * KSLOT]

    ys = (y0, y1, y2, y3)
    os_ = (o0, o1, o2, o3)
    base = s * RPT
    NSB = RPW_S // SB

    def idx_copies(sb, buf):
        r0 = s * RPW_S + sb * SB
        return (pltpu.make_async_copy(
                    src_hbm.at[pl.ds(r0, SB)], srcv.at[buf], isem),
                pltpu.make_async_copy(
                    dst_hbm.at[pl.ds(r0, SB)], dstv.at[buf], isem))

    def process(y_ref, out_ref):
        for d in idx_copies(0, 0):
            d.start()
        pltpu.sync_copy(y_ref.at[pl.ds(base, RPT)], acc.at[pl.ds(base, RPT)])
        plsc.subcore_barrier()

        def sbody(sb, _):
            buf = lax.rem(sb, 2)
            for d in idx_copies(sb, buf):
                d.wait()

            @pl.when(sb + 1 < NSB)
            def _():
                for d in idx_copies(sb + 1, 1 - buf):
                    d.start()

            sv = srcv.at[buf]
            dv = dstv.at[buf]
            gdescs = [
                pltpu.async_copy(y_ref.at[sv.at[v]], rows.at[v], gsems[v])
                for v in range(KSLOT)
            ]
            pending = [None] * KSLOT
            for b in range(SB):
                v = b % KSLOT
                if b >= 1:
                    pv = (b - 1) % KSLOT
                    pending[pv].wait()
                    nb = b - 1 + KSLOT
                    if nb < SB:
                        gdescs[pv] = pltpu.async_copy(
                            y_ref.at[sv.at[nb]], rows.at[pv], gsems[pv])
                gdescs[v].wait()
                pending[v] = pltpu.async_copy(
                    rows.at[v], acc.at[dv.at[b]], ssems[v], add=True)
            pending[(SB - 1) % KSLOT].wait()
            return 0

        lax.fori_loop(0, NSB, sbody, 0)
        plsc.subcore_barrier()
        pltpu.sync_copy(acc.at[pl.ds(base, RPT)], out_ref.at[pl.ds(base, RPT)])
        plsc.subcore_barrier()

    for cid in range(2):
        @pl.when(c == cid)
        def _():
            process(ys[2 * cid], os_[2 * cid])
            process(ys[2 * cid + 1], os_[2 * cid + 1])


def _dvec(degt_ref):
    deg = degt_ref[:, 0:1] + degt_ref[:, 1:2] + 1.0
    return lax.rsqrt(deg)


def _tc1_body(degt_ref, z_ref, w1_ref, y0_ref, y1_ref, y2_ref, y3_ref):
    d = _dvec(degt_ref)
    xw = jnp.dot(z_ref[...], w1_ref[...], preferred_element_type=jnp.float32)
    y = xw * d
    y0_ref[...] = y[:, 0:32]
    y1_ref[...] = y[:, 32:64]
    y2_ref[...] = y[:, 64:96]
    y3_ref[...] = y[:, 96:128]


def _tc2_body(degt_ref, s0_ref, s1_ref, s2_ref, s3_ref, b1_ref, w2_ref,
              y0_ref, y1_ref, y2_ref, y3_ref):
    d = _dvec(degt_ref)
    S = jnp.concatenate(
        [s0_ref[...], s1_ref[...], s2_ref[...], s3_ref[...]], axis=1)
    h = jnp.maximum(S * d + b1_ref[...], 0.0)
    xw = jnp.dot(h, w2_ref[...], preferred_element_type=jnp.float32)
    y = xw * d
    y0_ref[...] = y[:, 0:32]
    y1_ref[...] = y[:, 32:64]
    y2_ref[...] = y[:, 64:96]
    y3_ref[...] = y[:, 96:128]


def _tc3_body(degt_ref, s0_ref, s1_ref, s2_ref, s3_ref, b2_ref, out_ref):
    d = _dvec(degt_ref)
    S = jnp.concatenate(
        [s0_ref[...], s1_ref[...], s2_ref[...], s3_ref[...]], axis=1)
    out_ref[...] = jnp.maximum(S * d + b2_ref[...], 0.0)


_GRID = (NP // BR,)
_bs_degt = pl.BlockSpec((BR, 2), lambda i: (i, 0))
_bs_z = pl.BlockSpec((BR, IN_DIM), lambda i: (i, 0))
_bs_w1 = pl.BlockSpec((IN_DIM, FD), lambda i: (0, 0))
_bs_w2 = pl.BlockSpec((FD, FD), lambda i: (0, 0))
_bs_b = pl.BlockSpec((1, FD), lambda i: (0, 0))
_bs_c = pl.BlockSpec((BR, FC), lambda i: (i, 0))
_bs_f = pl.BlockSpec((BR, FD), lambda i: (i, 0))

_chunk4 = [jax.ShapeDtypeStruct((NP, FC), jnp.float32)] * 4

_tc1 = pl.pallas_call(
    _tc1_body, grid=_GRID,
    in_specs=[_bs_degt, _bs_z, _bs_w1],
    out_specs=[_bs_c] * 4,
    out_shape=_chunk4,
)

_tc2 = pl.pallas_call(
    _tc2_body, grid=_GRID,
    in_specs=[_bs_degt, _bs_c, _bs_c, _bs_c, _bs_c, _bs_b, _bs_w2],
    out_specs=[_bs_c] * 4,
    out_shape=_chunk4,
)

_tc3 = pl.pallas_call(
    _tc3_body, grid=_GRID,
    in_specs=[_bs_degt, _bs_c, _bs_c, _bs_c, _bs_c, _bs_b],
    out_specs=_bs_f,
    out_shape=jax.ShapeDtypeStruct((NP, FD), jnp.float32),
)


def kernel(z, edge_index, W1, b1, W2, b2):
    src = jnp.concatenate(
        [edge_index[0], jnp.zeros((EP - E,), jnp.int32)]).reshape(ROWS_E, 128)
    dst = jnp.concatenate(
        [edge_index[1], jnp.full((EP - E,), N, jnp.int32)]).reshape(ROWS_E, 128)
    z_pad = jnp.pad(z, ((0, NP - N), (0, 0)))

    degt = _deg_kernel(dst).reshape(2, NP).T

    y1 = _tc1(degt, z_pad, W1)
    s1 = _spmm_kernel(*y1, src, dst)
    y2 = _tc2(degt, *s1, b1.reshape(1, FD), W2)
    s2 = _spmm_kernel(*y2, src, dst)
    xp = _tc3(degt, *s2, b2.reshape(1, FD))
    return xp[:N]


# speedup vs baseline: 13.6070x; 1.1330x over previous
import functools

import jax
import jax.numpy as jnp
from jax import lax
from jax.experimental import pallas as pl
from jax.experimental.pallas import tpu as pltpu
from jax.experimental.pallas import tpu_sc as plsc

N = 50000
E = 800000
IN_DIM = 64
FD = 128
FC = 16
NCH = FD // FC
NP = 50176
EP = 819200
ROWS_E = EP // 128
RPW_H = ROWS_E // 32
RPW_S = ROWS_E // 16
RPT = NP // 16
SB = 16
KSLOT = 6
BR = 1568

_mesh = plsc.VectorSubcoreMesh(core_axis_name="c", subcore_axis_name="s")


@functools.partial(
    pl.kernel,
    out_type=jax.ShapeDtypeStruct((2 * NP,), jnp.float32),
    mesh=_mesh,
    compiler_params=pltpu.CompilerParams(needs_layout_passes=False),
    scratch_types=[
        pltpu.VMEM((40, 128), jnp.int32),
        pltpu.VMEM((NP,), jnp.float32),
        pltpu.VMEM_SHARED((16 * NP,), jnp.float32),
        pltpu.VMEM((RPT,), jnp.float32),
        pltpu.VMEM((RPT,), jnp.float32),
    ],
)
def _deg_kernel(dst_hbm, out_hbm, dstv, hist, shared, racc, rtmp):
    c = lax.axis_index("c")
    s = lax.axis_index("s")
    w = c * 16 + s

    zero16 = jnp.zeros((16,), jnp.float32)

    def zbody(i, _):
        hist[pl.ds(i * 16, 16)] = zero16
        return 0

    lax.fori_loop(0, NP // 16, zbody, 0)

    ones16 = jnp.ones((16,), jnp.float32)

    def kbody(k, _):
        pltpu.sync_copy(dst_hbm.at[pl.ds(w * RPW_H + k * 40, 40)], dstv)

        def hbody(r, _):
            for j in range(8):
                idx = dstv[r, pl.ds(j * 16, 16)]
                plsc.addupdate_scatter(hist, [idx], ones16)
            return 0

        lax.fori_loop(0, 40, hbody, 0)
        return 0

    lax.fori_loop(0, RPW_H // 40, kbody, 0)

    pltpu.sync_copy(hist, shared.at[pl.ds(s * NP, NP)])
    plsc.subcore_barrier()

    base = s * RPT
    pltpu.sync_copy(shared.at[pl.ds(base, RPT)], racc)
    for t in range(1, 16):
        pltpu.sync_copy(shared.at[pl.ds(t * NP + base, RPT)], rtmp)

        def abody(i, _):
            sl = pl.ds(i * 16, 16)
            racc[sl] = racc[sl] + rtmp[sl]
            return 0

        lax.fori_loop(0, RPT // 16, abody, 0)

    pltpu.sync_copy(racc, out_hbm.at[pl.ds(c * NP + base, RPT)])


@functools.partial(
    pl.kernel,
    out_type=[jax.ShapeDtypeStruct((NP, FC), jnp.float32)] * NCH,
    mesh=_mesh,
    compiler_params=pltpu.CompilerParams(
        needs_layout_passes=False, use_tc_tiling_on_sc=False),
    scratch_types=[
        pltpu.VMEM((2, SB, 128), jnp.int32),
        pltpu.VMEM((2, SB, 128), jnp.int32),
        pltpu.VMEM((KSLOT, 128, FC), jnp.float32),
        pltpu.VMEM_SHARED((NP, FC), jnp.float32),
        pltpu.VMEM_SHARED((NP, FC), jnp.float32),
    ] + [pltpu.SemaphoreType.DMA] * (2 * KSLOT + 1),
)
def _spmm_kernel(y0, y1, y2, y3, y4, y5, y6, y7, src_hbm, dst_hbm,
                 o0, o1, o2, o3, o4, o5, o6, o7,
                 srcv, dstv, rows, ysp, acc, *sems):
    c = lax.axis_index("c")
    s = lax.axis_index("s")
    gsems = sems[:KSLOT]
    ssems = sems[KSLOT:2 * KSLOT]
    isem = sems[2 * KSLOT]

    ys = (y0, y1, y2, y3, y4, y5, y6, y7)
    os_ = (o0, o1, o2, o3, o4, o5, o6, o7)
    base = s * RPT
    NSB = RPW_S // SB

    def idx_copies(sb, buf):
        r0 = s * RPW_S + sb * SB
        return (pltpu.make_async_copy(
                    src_hbm.at[pl.ds(r0, SB)], srcv.at[buf], isem),
                pltpu.make_async_copy(
                    dst_hbm.at[pl.ds(r0, SB)], dstv.at[buf], isem))

    def process(y_ref, out_ref):
        for d in idx_copies(0, 0):
            d.start()
        pltpu.sync_copy(y_ref.at[pl.ds(base, RPT)], ysp.at[pl.ds(base, RPT)])
        pltpu.sync_copy(y_ref.at[pl.ds(base, RPT)], acc.at[pl.ds(base, RPT)])
        plsc.subcore_barrier()

        def sbody(sb, _):
            buf = lax.rem(sb, 2)
            for d in idx_copies(sb, buf):
                d.wait()

            @pl.when(sb + 1 < NSB)
            def _():
                for d in idx_copies(sb + 1, 1 - buf):
                    d.start()

            sv = srcv.at[buf]
            dv = dstv.at[buf]
            gdescs = [
                pltpu.async_copy(ysp.at[sv.at[v]], rows.at[v], gsems[v])
                for v in range(KSLOT)
            ]
            pending = [None] * KSLOT
            for b in range(SB):
                v = b % KSLOT
                if b >= 1:
                    pv = (b - 1) % KSLOT
                    pending[pv].wait()
                    nb = b - 1 + KSLOT
                    if nb < SB:
                        gdescs[pv] = pltpu.async_copy(
                            ysp.at[sv.at[nb]], rows.at[pv], gsems[pv])
                gdescs[v].wait()
                pending[v] = pltpu.async_copy(
                    rows.at[v], acc.at[dv.at[b]], ssems[v], add=True)
            pending[(SB - 1) % KSLOT].wait()
            return 0

        lax.fori_loop(0, NSB, sbody, 0)
        plsc.subcore_barrier()
        pltpu.sync_copy(acc.at[pl.ds(base, RPT)], out_ref.at[pl.ds(base, RPT)])
        plsc.subcore_barrier()

    for cid in range(2):
        @pl.when(c == cid)
        def _():
            for j in range(NCH // 2):
                process(ys[(NCH // 2) * cid + j], os_[(NCH // 2) * cid + j])


def _dvec(degt_ref):
    deg = degt_ref[:, 0:1] + degt_ref[:, 1:2] + 1.0
    return lax.rsqrt(deg)


def _store_chunks(y, refs):
    for i, r in enumerate(refs):
        r[...] = y[:, i * FC:(i + 1) * FC]


def _tc1_body(degt_ref, z_ref, w1_ref, *y_refs):
    d = _dvec(degt_ref)
    xw = jnp.dot(z_ref[...], w1_ref[...], preferred_element_type=jnp.float32)
    _store_chunks(xw * d, y_refs)


def _tc2_body(degt_ref, s0, s1, s2, s3, s4, s5, s6, s7, b1_ref, w2_ref,
              *y_refs):
    d = _dvec(degt_ref)
    S = jnp.concatenate([r[...] for r in (s0, s1, s2, s3, s4, s5, s6, s7)],
                        axis=1)
    h = jnp.maximum(S * d + b1_ref[...], 0.0)
    xw = jnp.dot(h, w2_ref[...], preferred_element_type=jnp.float32)
    _store_chunks(xw * d, y_refs)


def _tc3_body(degt_ref, s0, s1, s2, s3, s4, s5, s6, s7, b2_ref, out_ref):
    d = _dvec(degt_ref)
    S = jnp.concatenate([r[...] for r in (s0, s1, s2, s3, s4, s5, s6, s7)],
                        axis=1)
    out_ref[...] = jnp.maximum(S * d + b2_ref[...], 0.0)


_GRID = (NP // BR,)
_bs_degt = pl.BlockSpec((BR, 2), lambda i: (i, 0))
_bs_z = pl.BlockSpec((BR, IN_DIM), lambda i: (i, 0))
_bs_w1 = pl.BlockSpec((IN_DIM, FD), lambda i: (0, 0))
_bs_w2 = pl.BlockSpec((FD, FD), lambda i: (0, 0))
_bs_b = pl.BlockSpec((1, FD), lambda i: (0, 0))
_bs_c = pl.BlockSpec((BR, FC), lambda i: (i, 0))
_bs_f = pl.BlockSpec((BR, FD), lambda i: (i, 0))

_chunks = [jax.ShapeDtypeStruct((NP, FC), jnp.float32)] * NCH

_tc1 = pl.pallas_call(
    _tc1_body, grid=_GRID,
    in_specs=[_bs_degt, _bs_z, _bs_w1],
    out_specs=[_bs_c] * NCH,
    out_shape=_chunks,
)

_tc2 = pl.pallas_call(
    _tc2_body, grid=_GRID,
    in_specs=[_bs_degt] + [_bs_c] * NCH + [_bs_b, _bs_w2],
    out_specs=[_bs_c] * NCH,
    out_shape=_chunks,
)

_tc3 = pl.pallas_call(
    _tc3_body, grid=_GRID,
    in_specs=[_bs_degt] + [_bs_c] * NCH + [_bs_b],
    out_specs=_bs_f,
    out_shape=jax.ShapeDtypeStruct((NP, FD), jnp.float32),
)


def kernel(z, edge_index, W1, b1, W2, b2):
    src = jnp.concatenate(
        [edge_index[0], jnp.zeros((EP - E,), jnp.int32)]).reshape(ROWS_E, 128)
    dst = jnp.concatenate(
        [edge_index[1], jnp.full((EP - E,), N, jnp.int32)]).reshape(ROWS_E, 128)
    z_pad = jnp.pad(z, ((0, NP - N), (0, 0)))

    degt = _deg_kernel(dst).reshape(2, NP).T

    y1 = _tc1(degt, z_pad, W1)
    s1 = _spmm_kernel(*y1, src, dst)
    y2 = _tc2(degt, *s1, b1.reshape(1, FD), W2)
    s2 = _spmm_kernel(*y2, src, dst)
    xp = _tc3(degt, *s2, b2.reshape(1, FD))
    return xp[:N]


# speedup vs baseline: 16.7155x; 1.2284x over previous
import functools

import jax
import jax.numpy as jnp
from jax import lax
from jax.experimental import pallas as pl
from jax.experimental.pallas import tpu as pltpu
from jax.experimental.pallas import tpu_sc as plsc

N = 50000
E = 800000
IN_DIM = 64
FD = 128
FC = 16
NCH = FD // FC
NP = 50176
EP = 819200
ROWS_E = EP // 128
RPW_H = ROWS_E // 32
RPW_S = ROWS_E // 16
RPT = NP // 16
SB = 16
KSLOT = 6
BR = 1568

_mesh = plsc.VectorSubcoreMesh(core_axis_name="c", subcore_axis_name="s")


@functools.partial(
    pl.kernel,
    out_type=jax.ShapeDtypeStruct((2 * NP,), jnp.float32),
    mesh=_mesh,
    compiler_params=pltpu.CompilerParams(needs_layout_passes=False),
    scratch_types=[
        pltpu.VMEM((40, 128), jnp.int32),
        pltpu.VMEM((NP,), jnp.float32),
        pltpu.VMEM_SHARED((16 * NP,), jnp.float32),
        pltpu.VMEM((RPT,), jnp.float32),
        pltpu.VMEM((RPT,), jnp.float32),
    ],
)
def _deg_kernel(dst_hbm, out_hbm, dstv, hist, shared, racc, rtmp):
    c = lax.axis_index("c")
    s = lax.axis_index("s")
    w = c * 16 + s

    zero16 = jnp.zeros((16,), jnp.float32)

    def zbody(i, _):
        hist[pl.ds(i * 16, 16)] = zero16
        return 0

    lax.fori_loop(0, NP // 16, zbody, 0)

    ones16 = jnp.ones((16,), jnp.float32)

    def kbody(k, _):
        pltpu.sync_copy(dst_hbm.at[pl.ds(w * RPW_H + k * 40, 40)], dstv)

        def hbody(r, _):
            for j in range(8):
                idx = dstv[r, pl.ds(j * 16, 16)]
                plsc.addupdate_scatter(hist, [idx], ones16)
            return 0

        lax.fori_loop(0, 40, hbody, 0)
        return 0

    lax.fori_loop(0, RPW_H // 40, kbody, 0)

    pltpu.sync_copy(hist, shared.at[pl.ds(s * NP, NP)])
    plsc.subcore_barrier()

    base = s * RPT
    pltpu.sync_copy(shared.at[pl.ds(base, RPT)], racc)
    for t in range(1, 16):
        pltpu.sync_copy(shared.at[pl.ds(t * NP + base, RPT)], rtmp)

        def abody(i, _):
            sl = pl.ds(i * 16, 16)
            racc[sl] = racc[sl] + rtmp[sl]
            return 0

        lax.fori_loop(0, RPT // 16, abody, 0)

    pltpu.sync_copy(racc, out_hbm.at[pl.ds(c * NP + base, RPT)])


@functools.partial(
    pl.kernel,
    out_type=jax.ShapeDtypeStruct((NP, FD), jnp.float32),
    mesh=_mesh,
    compiler_params=pltpu.CompilerParams(
        needs_layout_passes=False, use_tc_tiling_on_sc=False),
    scratch_types=[
        pltpu.VMEM((2, SB, 128), jnp.int32),
        pltpu.VMEM((2, SB, 128), jnp.int32),
        pltpu.VMEM((KSLOT, 128, FC), jnp.float32),
        pltpu.VMEM_SHARED((NP, FC), jnp.float32),
        pltpu.VMEM_SHARED((NP, FC), jnp.float32),
    ] + [pltpu.SemaphoreType.DMA] * (2 * KSLOT + 1),
)
def _spmm_kernel(y_hbm, src_hbm, dst_hbm, out_hbm,
                 srcv, dstv, rows, ysp, acc, *sems):
    c = lax.axis_index("c")
    s = lax.axis_index("s")
    gsems = sems[:KSLOT]
    ssems = sems[KSLOT:2 * KSLOT]
    isem = sems[2 * KSLOT]

    base = s * RPT
    rsl = pl.ds(base, RPT)
    NSB = RPW_S // SB

    def idx_copies(sb, buf):
        r0 = s * RPW_S + sb * SB
        return (pltpu.make_async_copy(
                    src_hbm.at[pl.ds(r0, SB)], srcv.at[buf], isem),
                pltpu.make_async_copy(
                    dst_hbm.at[pl.ds(r0, SB)], dstv.at[buf], isem))

    def process(ch):
        csl = pl.ds(ch * FC, FC)
        for d in idx_copies(0, 0):
            d.start()
        pltpu.sync_copy(y_hbm.at[rsl, csl], ysp.at[rsl])
        pltpu.sync_copy(y_hbm.at[rsl, csl], acc.at[rsl])
        plsc.subcore_barrier()

        def sbody(sb, _):
            buf = lax.rem(sb, 2)
            for d in idx_copies(sb, buf):
                d.wait()

            @pl.when(sb + 1 < NSB)
            def _():
                for d in idx_copies(sb + 1, 1 - buf):
                    d.start()

            sv = srcv.at[buf]
            dv = dstv.at[buf]
            gdescs = [
                pltpu.async_copy(ysp.at[sv.at[v]], rows.at[v], gsems[v])
                for v in range(KSLOT)
            ]
            pending = [None] * KSLOT
            for b in range(SB):
                v = b % KSLOT
                if b >= 1:
                    pv = (b - 1) % KSLOT
                    pending[pv].wait()
                    nb = b - 1 + KSLOT
                    if nb < SB:
                        gdescs[pv] = pltpu.async_copy(
                            ysp.at[sv.at[nb]], rows.at[pv], gsems[pv])
                gdescs[v].wait()
                pending[v] = pltpu.async_copy(
                    rows.at[v], acc.at[dv.at[b]], ssems[v], add=True)
            pending[(SB - 1) % KSLOT].wait()
            return 0

        lax.fori_loop(0, NSB, sbody, 0)
        plsc.subcore_barrier()
        pltpu.sync_copy(acc.at[rsl], out_hbm.at[rsl, csl])
        plsc.subcore_barrier()

    for cid in range(2):
        @pl.when(c == cid)
        def _():
            for j in range(NCH // 2):
                process((NCH // 2) * cid + j)


def _dvec(degt_ref):
    deg = degt_ref[:, 0:1] + degt_ref[:, 1:2] + 1.0
    return lax.rsqrt(deg)


def _tc1_body(degt_ref, z_ref, w1_ref, y_ref):
    d = _dvec(degt_ref)
    xw = jnp.dot(z_ref[...], w1_ref[...], preferred_element_type=jnp.float32)
    y_ref[...] = xw * d


def _tc2_body(degt_ref, s_ref, b1_ref, w2_ref, y_ref):
    d = _dvec(degt_ref)
    h = jnp.maximum(s_ref[...] * d + b1_ref[...], 0.0)
    xw = jnp.dot(h, w2_ref[...], preferred_element_type=jnp.float32)
    y_ref[...] = xw * d


def _tc3_body(degt_ref, s_ref, b2_ref, out_ref):
    d = _dvec(degt_ref)
    out_ref[...] = jnp.maximum(s_ref[...] * d + b2_ref[...], 0.0)


_GRID = (NP // BR,)
_bs_degt = pl.BlockSpec((BR, 2), lambda i: (i, 0))
_bs_z = pl.BlockSpec((BR, IN_DIM), lambda i: (i, 0))
_bs_w1 = pl.BlockSpec((IN_DIM, FD), lambda i: (0, 0))
_bs_w2 = pl.BlockSpec((FD, FD), lambda i: (0, 0))
_bs_b = pl.BlockSpec((1, FD), lambda i: (0, 0))
_bs_f = pl.BlockSpec((BR, FD), lambda i: (i, 0))

_full = jax.ShapeDtypeStruct((NP, FD), jnp.float32)

_tc1 = pl.pallas_call(
    _tc1_body, grid=_GRID,
    in_specs=[_bs_degt, _bs_z, _bs_w1],
    out_specs=_bs_f,
    out_shape=_full,
)

_tc2 = pl.pallas_call(
    _tc2_body, grid=_GRID,
    in_specs=[_bs_degt, _bs_f, _bs_b, _bs_w2],
    out_specs=_bs_f,
    out_shape=_full,
)

_tc3 = pl.pallas_call(
    _tc3_body, grid=_GRID,
    in_specs=[_bs_degt, _bs_f, _bs_b],
    out_specs=_bs_f,
    out_shape=_full,
)


def kernel(z, edge_index, W1, b1, W2, b2):
    src = jnp.concatenate(
        [edge_index[0], jnp.zeros((EP - E,), jnp.int32)]).reshape(ROWS_E, 128)
    dst = jnp.concatenate(
        [edge_index[1], jnp.full((EP - E,), N, jnp.int32)]).reshape(ROWS_E, 128)
    z_pad = jnp.pad(z, ((0, NP - N), (0, 0)))

    degt = _deg_kernel(dst).reshape(2, NP).T

    y1 = _tc1(degt, z_pad, W1)
    s1 = _spmm_kernel(y1, src, dst)
    y2 = _tc2(degt, s1, b1.reshape(1, FD), W2)
    s2 = _spmm_kernel(y2, src, dst)
    xp = _tc3(degt, s2, b2.reshape(1, FD))
    return xp[:N]
